# Initial kernel scaffold; baseline (speedup 1.0000x reference)
import jax
import jax.numpy as jnp
from jax.experimental import pallas as pl


def kernel(x, codebook):
    raise NotImplementedError("write your pallas kernel here")


# speedup vs baseline: 93.3912x; 93.3912x over previous
import functools

import jax
import jax.numpy as jnp
from jax import lax
from jax.experimental import pallas as pl
from jax.experimental.pallas import tpu as pltpu
from jax.experimental.pallas import tpu_sc as plsc

DIM = 256
NUM_CODES = 8192

TM = 512
TN = 1024


def _argmin_body(x_ref, cb_ref, ids_ref, min_ref, arg_ref):
    j = pl.program_id(1)
    nj = pl.num_programs(1)

    @pl.when(j == 0)
    def _init():
        min_ref[...] = jnp.full((TM, 1), jnp.inf, jnp.float32)
        arg_ref[...] = jnp.zeros((TM, 1), jnp.int32)

    x = x_ref[...]
    cb = cb_ref[...]
    scores = lax.dot_general(
        x, cb, (((1,), (1,)), ((), ())),
        preferred_element_type=jnp.float32,
        precision=lax.Precision.DEFAULT)
    x2 = jnp.sum(x * x, axis=1, keepdims=True)
    c2 = jnp.sum(cb * cb, axis=1)[None, :]
    d = jnp.maximum((x2 + c2) - 2.0 * scores, 0.0)

    row_min = jnp.min(d, axis=1, keepdims=True)
    col = lax.broadcasted_iota(jnp.int32, (TM, TN), 1) + j * TN
    big = jnp.int32(2**30)
    row_arg = jnp.min(jnp.where(d == row_min, col, big), axis=1,
                      keepdims=True)

    cur_min = min_ref[...]
    better = row_min < cur_min
    arg_ref[...] = jnp.where(better, row_arg, arg_ref[...])
    min_ref[...] = jnp.where(better, row_min, cur_min)

    @pl.when(j == nj - 1)
    def _emit():
        ids_ref[...] = arg_ref[...]


def _nearest_code_ids(x2d, codebook):
    m = x2d.shape[0]
    grid = (m // TM, NUM_CODES // TN)
    return pl.pallas_call(
        _argmin_body,
        grid=grid,
        in_specs=[
            pl.BlockSpec((TM, DIM), lambda i, j: (i, 0)),
            pl.BlockSpec((TN, DIM), lambda i, j: (j, 0)),
        ],
        out_specs=pl.BlockSpec((TM, 1), lambda i, j: (i, 0)),
        out_shape=jax.ShapeDtypeStruct((m, 1), jnp.int32),
        scratch_shapes=[
            pltpu.VMEM((TM, 1), jnp.float32),
            pltpu.VMEM((TM, 1), jnp.int32),
        ],
        compiler_params=pltpu.CompilerParams(
            dimension_semantics=("parallel", "arbitrary")),
    )(x2d, codebook)


def _make_sc_gather(n_rows):
    info = plsc.get_sparse_core_info()
    nw = info.num_cores * info.num_subcores
    per_w = n_rows // nw
    nc = info.num_cores

    @functools.partial(
        pl.kernel,
        out_type=jax.ShapeDtypeStruct((n_rows, DIM), jnp.float32),
        mesh=plsc.VectorSubcoreMesh(core_axis_name="c", subcore_axis_name="s"),
        scratch_types=[
            pltpu.VMEM((per_w,), jnp.int32),
            pltpu.VMEM((per_w, DIM), jnp.float32),
            pltpu.SemaphoreType.DMA,
        ],
    )
    def gather(table_hbm, idx_hbm, out_hbm, idx_v, rows_v, sem):
        wid = lax.axis_index("s") * nc + lax.axis_index("c")
        base = wid * per_w
        pltpu.sync_copy(idx_hbm.at[pl.ds(base, per_w)], idx_v)
        pltpu.async_copy(table_hbm.at[idx_v], rows_v, sem).wait()
        pltpu.sync_copy(rows_v, out_hbm.at[pl.ds(base, per_w)])

    return gather


def kernel(x, codebook):
    b, t, dim = x.shape
    m = b * t
    x2d = x.reshape(m, dim)
    ids2d = _nearest_code_ids(x2d, codebook)
    ids_flat = ids2d.reshape(m)
    outputs = _make_sc_gather(m)(codebook, ids_flat)
    return (outputs.reshape(b, t, dim),
            ids2d.reshape(b, t, 1).astype(jnp.int64))
